# baseline (speedup 1.0000x reference)
import math
from functools import partial

import jax
import jax.numpy as jnp
from jax import lax
from jax.experimental import pallas as pl
from jax.experimental.pallas import tpu as pltpu
from jax.experimental.pallas import tpu_sc as plsc

_H = 8
_NEG = -1e30
_LANES = 16


def _attn_kernel(ctt_ref, wq_ref, bq_ref, wk_ref, bk_ref, wv_ref, bv_ref,
                 xq_ref, xk_ref, xv_ref, out_ref, ct_scratch, *, n_top, bs):
    n = ctt_ref.shape[0]
    d = wq_ref.shape[0]
    dh = d // _H
    f32 = jnp.float32

    eye = (lax.broadcasted_iota(jnp.int32, (n, n), 0)
           == lax.broadcasted_iota(jnp.int32, (n, n), 1)).astype(f32)

    ct_scratch[...] = lax.dot_general(
        ctt_ref[...], eye, (((0,), (0,)), ((), ())),
        preferred_element_type=f32)

    ct = ct_scratch[...]
    ct_pos = ct > 0.0

    def proj(x, w_ref, b_ref):
        return lax.dot_general(x, w_ref[...],
                               (((1,), (0,)), ((), ())),
                               preferred_element_type=f32) + b_ref[...]

    ones_col = jnp.ones((n, 1), f32)
    inv_scale = 1.0 / math.sqrt(dh)
    m_rows = []
    ctxs = []
    vmeans = []
    q_all = proj(xq_ref[...].reshape(bs * n, d), wq_ref, bq_ref)
    k_all = proj(xk_ref[...].reshape(bs * n, d), wk_ref, bk_ref)
    v_all = proj(xv_ref[...].reshape(bs * n, d), wv_ref, bv_ref)
    for sl in range(bs):
        q = q_all[sl * n:(sl + 1) * n]
        k = k_all[sl * n:(sl + 1) * n]
        v = v_all[sl * n:(sl + 1) * n]
        for h in range(_H):
            qh = q[:, h * dh:(h + 1) * dh]
            kh = k[:, h * dh:(h + 1) * dh]
            vh = v[:, h * dh:(h + 1) * dh]
            s = lax.dot_general(qh, kh, (((1,), (1,)), ((), ())),
                                preferred_element_type=f32)
            s_t = lax.dot_general(kh, qh, (((1,), (1,)), ((), ())),
                                  preferred_element_type=f32)
            m_max = jnp.max(jnp.where(ct_pos, s_t, _NEG), axis=0,
                            keepdims=True)
            qT = lax.dot_general(qh, eye, (((0,), (0,)), ((), ())),
                                 preferred_element_type=f32)
            w_kc = lax.dot_general(kh, ct, (((0,), (0,)), ((), ())),
                                   preferred_element_type=f32)
            m_sum = jnp.sum(qT * w_kc, axis=0, keepdims=True)
            m_rows.append(m_max - m_sum * (1.0 / n))

            sc = s * inv_scale
            cmax = jnp.max(sc, axis=1, keepdims=True)
            e = jnp.exp(sc - cmax)
            denom = lax.dot_general(e, ones_col, (((1,), (0,)), ((), ())),
                                    preferred_element_type=f32)
            ctx_u = lax.dot_general(e, vh, (((1,), (0,)), ((), ())),
                                    preferred_element_type=f32)
            ctxs.append(ctx_u * (1.0 / denom))
            vmeans.append(jnp.mean(vh, axis=0, keepdims=True))

    nrow = bs * _H
    m_all = jnp.concatenate(m_rows, axis=0)
    work = m_all
    for _ in range(n_top - 1):
        mx = jnp.max(work, axis=1, keepdims=True)
        work = jnp.where(work == mx, _NEG, work)
    thr = jnp.max(work, axis=1, keepdims=True)
    sel = (m_all >= thr).astype(f32)

    sel_cols = lax.dot_general(eye, sel, (((1,), (1,)), ((), ())),
                               preferred_element_type=f32)

    for sl in range(bs):
        for h in range(_H):
            r = sl * _H + h
            out_ref[0, sl, :, h * dh:(h + 1) * dh] = jnp.where(
                sel_cols[:, r:r + 1] > 0.5, ctxs[r], vmeans[r])


def _sc_counts_body(nq_per, mp, sk, ncores, flat_hbm, out_hbm, idx_v, acc_v):
    c = lax.axis_index("c")
    s = lax.axis_index("s")
    w = s * ncores + c
    idx_row = idx_v.shape[0]
    blk = nq_per * mp
    pltpu.sync_copy(flat_hbm.at[pl.ds(w * idx_row, idx_row)], idx_v)
    nchunks = mp // _LANES
    base = lax.broadcasted_iota(jnp.int32, (_LANES,), 0)
    zeros = jnp.zeros((_LANES,), jnp.float32)

    def zbody(i, carry):
        acc_v[pl.ds(i * _LANES, _LANES)] = zeros
        return carry

    lax.fori_loop(0, blk // _LANES, zbody, 0)
    for ql in range(nq_per):
        def body(si, carry):
            ivec = idx_v[pl.ds(ql * sk + si, _LANES)]
            bvec = jnp.full((_LANES,), ivec[0], jnp.int32)
            for ci in range(nchunks):
                sl = pl.ds(ql * mp + ci * _LANES, _LANES)
                diff = jnp.abs((base + ci * _LANES) - bvec)
                hit = (1 - jnp.minimum(diff, 1)).astype(jnp.float32)
                acc_v[sl] = acc_v[sl] + hit
            return carry

        lax.fori_loop(0, sk, body, 0)
    pltpu.sync_copy(acc_v, out_hbm.at[pl.ds(w * blk, blk)])


def _build_counts(index_sample, n):
    nq, sk = index_sample.shape
    info = plsc.get_sparse_core_info()
    nw = info.num_cores * info.num_subcores
    nq_per = -(-nq // nw)
    mp = -(-n // _LANES) * _LANES
    idx_row = (-(-(nq_per * sk) // _LANES) + 1) * _LANES
    idx_pad = jnp.zeros((nw * nq_per, sk), jnp.int32)
    idx_pad = idx_pad.at[:nq].set(index_sample.astype(jnp.int32))
    rows = idx_pad.reshape(nw, nq_per * sk)
    rows = jnp.pad(rows, ((0, 0), (0, idx_row - nq_per * sk))).reshape(-1)
    mesh = plsc.VectorSubcoreMesh(core_axis_name="c", subcore_axis_name="s")
    out = pl.kernel(
        partial(_sc_counts_body, nq_per, mp, sk, info.num_cores),
        out_type=jax.ShapeDtypeStruct((nw * nq_per * mp,), jnp.float32),
        mesh=mesh,
        scratch_types=[
            pltpu.VMEM((idx_row,), jnp.int32),
            pltpu.VMEM((nq_per * mp,), jnp.float32),
        ],
    )(rows)
    return out.reshape(nw * nq_per, mp)[:nq, :n]


def kernel(queries, keys, values, Wq, bq, Wk, bk, Wv, bv, index_sample):
    b, l, n, d = queries.shape
    n_top = min(index_sample.shape[1], n)
    ct = _build_counts(index_sample, n)
    bq2 = bq.reshape(1, d)
    bk2 = bk.reshape(1, d)
    bv2 = bv.reshape(1, d)
    wqt = Wq.T
    wkt = Wk.T
    wvt = Wv.T

    bs = l
    fixed = lambda i: (0, 0)
    sliced = lambda i: (i, 0, 0, 0)
    out = pl.pallas_call(
        partial(_attn_kernel, n_top=n_top, bs=bs),
        grid=(b,),
        in_specs=[
            pl.BlockSpec((n, n), fixed),
            pl.BlockSpec((d, d), fixed),
            pl.BlockSpec((1, d), fixed),
            pl.BlockSpec((d, d), fixed),
            pl.BlockSpec((1, d), fixed),
            pl.BlockSpec((d, d), fixed),
            pl.BlockSpec((1, d), fixed),
            pl.BlockSpec((1, bs, n, d), sliced),
            pl.BlockSpec((1, bs, n, d), sliced),
            pl.BlockSpec((1, bs, n, d), sliced),
        ],
        out_specs=pl.BlockSpec((1, bs, n, d), sliced),
        out_shape=jax.ShapeDtypeStruct((b, l, n, d), jnp.float32),
        scratch_shapes=[pltpu.VMEM((n, n), jnp.float32)],
        compiler_params=pltpu.CompilerParams(
            dimension_semantics=("parallel",)),
    )(ct, wqt, bq2, wkt, bk2, wvt, bv2, queries, keys, values)
    return out


# speedup vs baseline: 1.2222x; 1.2222x over previous
import math
from functools import partial

import jax
import jax.numpy as jnp
from jax import lax
from jax.experimental import pallas as pl
from jax.experimental.pallas import tpu as pltpu
from jax.experimental.pallas import tpu_sc as plsc

_H = 8
_NEG = -1e30
_LANES = 16


def _attn_kernel(ctt_ref, wq_ref, bq_ref, wk_ref, bk_ref, wv_ref, bv_ref,
                 xq_ref, xk_ref, xv_ref, out_ref, ct_scratch, *, n_top, bs):
    n = ctt_ref.shape[0]
    d = wq_ref.shape[0]
    dh = d // _H
    f32 = jnp.float32

    eye = (lax.broadcasted_iota(jnp.int32, (n, n), 0)
           == lax.broadcasted_iota(jnp.int32, (n, n), 1)).astype(f32)

    ct_scratch[...] = lax.dot_general(
        ctt_ref[...], eye, (((0,), (0,)), ((), ())),
        preferred_element_type=f32)

    ct = ct_scratch[...]
    ct_pos = ct > 0.0

    def proj(x, w_ref, b_ref):
        return lax.dot_general(x, w_ref[...],
                               (((1,), (0,)), ((), ())),
                               preferred_element_type=f32) + b_ref[...]

    ones_col = jnp.ones((n, 1), f32)
    inv_scale = 1.0 / math.sqrt(dh)
    m_rows = []
    ctxs = []
    vmeans = []
    q_all = proj(xq_ref[...].reshape(bs * n, d), wq_ref, bq_ref)
    k_all = proj(xk_ref[...].reshape(bs * n, d), wk_ref, bk_ref)
    v_all = proj(xv_ref[...].reshape(bs * n, d), wv_ref, bv_ref)
    for sl in range(bs):
        q = q_all[sl * n:(sl + 1) * n]
        k = k_all[sl * n:(sl + 1) * n]
        v = v_all[sl * n:(sl + 1) * n]
        for h in range(_H):
            qh = q[:, h * dh:(h + 1) * dh]
            kh = k[:, h * dh:(h + 1) * dh]
            vh = v[:, h * dh:(h + 1) * dh]
            s = lax.dot_general(qh, kh, (((1,), (1,)), ((), ())),
                                preferred_element_type=f32)
            s_t = lax.dot_general(kh, qh, (((1,), (1,)), ((), ())),
                                  preferred_element_type=f32)
            m_max = jnp.max(jnp.where(ct_pos, s_t, _NEG), axis=0,
                            keepdims=True)
            qT = lax.dot_general(qh, eye, (((0,), (0,)), ((), ())),
                                 preferred_element_type=f32)
            w_kc = lax.dot_general(kh, ct, (((0,), (0,)), ((), ())),
                                   preferred_element_type=f32)
            m_sum = jnp.sum(qT * w_kc, axis=0, keepdims=True)
            m_rows.append(m_max - m_sum * (1.0 / n))

            sc = s * inv_scale
            cmax = jnp.max(sc, axis=1, keepdims=True)
            e = jnp.exp(sc - cmax)
            v1 = jnp.concatenate([vh, ones_col], axis=1)
            cd = lax.dot_general(e, v1, (((1,), (0,)), ((), ())),
                                 preferred_element_type=f32)
            ctxs.append(cd[:, :dh] * (1.0 / cd[:, dh:dh + 1]))
            vmeans.append(jnp.mean(vh, axis=0, keepdims=True))

    nrow = bs * _H
    m_all = jnp.concatenate(m_rows, axis=0)
    work = m_all
    for _ in range(n_top - 1):
        mx = jnp.max(work, axis=1, keepdims=True)
        work = jnp.where(work == mx, _NEG, work)
    thr = jnp.max(work, axis=1, keepdims=True)
    sel = (m_all >= thr).astype(f32)

    sel_cols = lax.dot_general(eye, sel, (((1,), (1,)), ((), ())),
                               preferred_element_type=f32)

    for sl in range(bs):
        for h in range(_H):
            r = sl * _H + h
            out_ref[sl, :, h * dh:(h + 1) * dh] = jnp.where(
                sel_cols[:, r:r + 1] > 0.5, ctxs[r], vmeans[r])


def _sc_counts_body(nq_per, mp, sk, ncores, flat_hbm, out_hbm, idx_v, acc_v):
    c = lax.axis_index("c")
    s = lax.axis_index("s")
    w = s * ncores + c
    idx_row = idx_v.shape[0]
    blk = nq_per * mp
    pltpu.sync_copy(flat_hbm.at[pl.ds(w * idx_row, idx_row)], idx_v)
    nchunks = mp // _LANES
    base = lax.broadcasted_iota(jnp.int32, (_LANES,), 0)
    zeros = jnp.zeros((_LANES,), jnp.float32)

    def zbody(i, carry):
        acc_v[pl.ds(i * _LANES, _LANES)] = zeros
        return carry

    lax.fori_loop(0, blk // _LANES, zbody, 0)
    for ql in range(nq_per):
        def body(si, carry):
            ivec = idx_v[pl.ds(ql * sk + si, _LANES)]
            bvec = jnp.full((_LANES,), ivec[0], jnp.int32)
            for ci in range(nchunks):
                sl = pl.ds(ql * mp + ci * _LANES, _LANES)
                diff = jnp.abs((base + ci * _LANES) - bvec)
                hit = (1 - jnp.minimum(diff, 1)).astype(jnp.float32)
                acc_v[sl] = acc_v[sl] + hit
            return carry

        lax.fori_loop(0, sk, body, 0)
    pltpu.sync_copy(acc_v, out_hbm.at[pl.ds(w * blk, blk)])


def _build_counts(index_sample, n):
    nq, sk = index_sample.shape
    info = plsc.get_sparse_core_info()
    nw = info.num_cores * info.num_subcores
    nq_per = -(-nq // nw)
    mp = -(-n // _LANES) * _LANES
    idx_row = (-(-(nq_per * sk) // _LANES) + 1) * _LANES
    idx_pad = jnp.zeros((nw * nq_per, sk), jnp.int32)
    idx_pad = idx_pad.at[:nq].set(index_sample.astype(jnp.int32))
    rows = idx_pad.reshape(nw, nq_per * sk)
    rows = jnp.pad(rows, ((0, 0), (0, idx_row - nq_per * sk))).reshape(-1)
    mesh = plsc.VectorSubcoreMesh(core_axis_name="c", subcore_axis_name="s")
    out = pl.kernel(
        partial(_sc_counts_body, nq_per, mp, sk, info.num_cores),
        out_type=jax.ShapeDtypeStruct((nw * nq_per * mp,), jnp.float32),
        mesh=mesh,
        scratch_types=[
            pltpu.VMEM((idx_row,), jnp.int32),
            pltpu.VMEM((nq_per * mp,), jnp.float32),
        ],
    )(rows)
    return out.reshape(nw * nq_per, mp)[:nq, :n]


def kernel(queries, keys, values, Wq, bq, Wk, bk, Wv, bv, index_sample):
    b, l, n, d = queries.shape
    n_top = min(index_sample.shape[1], n)
    g = b * l
    ct = _build_counts(index_sample, n)
    xq = queries.reshape(g, n, d)
    xk = keys.reshape(g, n, d)
    xv = values.reshape(g, n, d)
    bq2 = bq.reshape(1, d)
    bk2 = bk.reshape(1, d)
    bv2 = bv.reshape(1, d)
    wqt = Wq.T
    wkt = Wk.T
    wvt = Wv.T

    bs = 8 if g % 8 == 0 else (4 if g % 4 == 0 else 1)
    fixed = lambda i: (0, 0)
    sliced = lambda i: (i, 0, 0)
    out = pl.pallas_call(
        partial(_attn_kernel, n_top=n_top, bs=bs),
        grid=(g // bs,),
        in_specs=[
            pl.BlockSpec((n, n), fixed),
            pl.BlockSpec((d, d), fixed),
            pl.BlockSpec((1, d), fixed),
            pl.BlockSpec((d, d), fixed),
            pl.BlockSpec((1, d), fixed),
            pl.BlockSpec((d, d), fixed),
            pl.BlockSpec((1, d), fixed),
            pl.BlockSpec((bs, n, d), sliced),
            pl.BlockSpec((bs, n, d), sliced),
            pl.BlockSpec((bs, n, d), sliced),
        ],
        out_specs=pl.BlockSpec((bs, n, d), sliced),
        out_shape=jax.ShapeDtypeStruct((g, n, d), jnp.float32),
        scratch_shapes=[pltpu.VMEM((n, n), jnp.float32)],
        compiler_params=pltpu.CompilerParams(
            dimension_semantics=("parallel",)),
    )(ct, wqt, bq2, wkt, bk2, wvt, bv2, xq, xk, xv)
    return out.reshape(b, l, n, d)
